# Initial kernel scaffold; baseline (speedup 1.0000x reference)
import jax
import jax.numpy as jnp
from jax.experimental import pallas as pl


def kernel(user_emb, item_emb, adj_indices, adj_values):
    raise NotImplementedError("write your pallas kernel here")


# speedup vs baseline: 2.3915x; 2.3915x over previous
import functools

import jax
import jax.numpy as jnp
from jax import lax
from jax.experimental import pallas as pl
from jax.experimental.pallas import tpu as pltpu
from jax.experimental.pallas import tpu_sc as plsc

_EMB = 64
_HALF = 25000
_PAD_HALF = 25088
_EGO_PAD = 2 * _PAD_HALF
_NS = 16
_CHUNK = 128
_SUB = 2
_SUPER = _CHUNK * _SUB
_ACC_SLICE = _PAD_HALF // _NS


def _spmm_body(ego, cols2, rows2, vals2, out, colv, rowv, rloc, valv, gath, acc, sem):
    c = lax.axis_index("c")
    s = lax.axis_index("s")
    per_tile = cols2.shape[0] // _NS
    steps = per_tile // _SUB

    zeros = jnp.zeros((16,), jnp.float32)

    def _zero_gath(i, carry):
        for q in range(_EMB // 16):
            gath[i, pl.ds(q * 16, 16)] = zeros
        return carry

    lax.fori_loop(0, _SUPER, _zero_gath, 0, unroll=8)

    a0 = s * _ACC_SLICE
    for z in range(6):
        pltpu.sync_copy(gath.at[pl.ds(0, _SUPER)],
                        acc.at[pl.ds(a0 + z * _SUPER, _SUPER)])
    pltpu.sync_copy(gath.at[pl.ds(0, _ACC_SLICE - 6 * _SUPER)],
                    acc.at[pl.ds(a0 + 6 * _SUPER, _ACC_SLICE - 6 * _SUPER)])
    plsc.subcore_barrier()

    c0 = c * _HALF

    def _step(g, carry):
        r0 = s * per_tile + g * _SUB
        pltpu.sync_copy(cols2.at[pl.ds(r0, _SUB)], colv)
        pltpu.sync_copy(rows2.at[pl.ds(r0, _SUB)], rowv)
        pltpu.sync_copy(vals2.at[pl.ds(r0, _SUB)], valv)

        for i in range(_SUB):
            for jj in range(_CHUNK // 16):
                sl = pl.ds(jj * 16, 16)
                r = rowv[i, sl]
                rl = r - c0
                ok = (rl >= 0) & (rl < _HALF)
                trash = _HALF + (r & 63)
                rloc[i, sl] = jnp.where(ok, rl, trash)

        descs = [
            pltpu.async_copy(ego.at[colv.at[j]],
                             gath.at[pl.ds(j * _CHUNK, _CHUNK)], sem)
            for j in range(_SUB)
        ]
        for d in descs:
            d.wait()

        for j in range(_SUB):
            def _scale(i, carry2, j=j):
                vv = valv[j, pl.ds(i * 16, 16)]
                for k in range(16):
                    v = vv[k]
                    row = j * _CHUNK + i * 16 + k
                    for q in range(_EMB // 16):
                        sl = pl.ds(q * 16, 16)
                        gath[row, sl] = gath[row, sl] * v
                return carry2

            lax.fori_loop(0, _CHUNK // 16, _scale, 0)
            pltpu.sync_copy(gath.at[pl.ds(j * _CHUNK, _CHUNK)],
                            acc.at[rloc.at[j]], add=True)
        return carry

    lax.fori_loop(0, steps, _step, 0)

    plsc.subcore_barrier()
    o0 = c * _PAD_HALF + a0
    h = _ACC_SLICE // 2
    pltpu.sync_copy(acc.at[pl.ds(a0, h)], out.at[pl.ds(o0, h)])
    pltpu.sync_copy(acc.at[pl.ds(a0 + h, h)], out.at[pl.ds(o0 + h, h)])


@functools.cache
def _make_layer():
    mesh = plsc.VectorSubcoreMesh(core_axis_name="c", subcore_axis_name="s")
    return pl.kernel(
        _spmm_body,
        out_type=jax.ShapeDtypeStruct((_EGO_PAD, _EMB), jnp.float32),
        mesh=mesh,
        compiler_params=pltpu.CompilerParams(use_tc_tiling_on_sc=False),
        scratch_types=[
            pltpu.VMEM((_SUB, _CHUNK), jnp.int32),
            pltpu.VMEM((_SUB, _CHUNK), jnp.int32),
            pltpu.VMEM((_SUB, _CHUNK), jnp.int32),
            pltpu.VMEM((_SUB, _CHUNK), jnp.float32),
            pltpu.VMEM((_SUPER, _EMB), jnp.float32),
            pltpu.VMEM_SHARED((_PAD_HALF, _EMB), jnp.float32),
            pltpu.SemaphoreType.DMA,
        ],
    )


def kernel(user_emb, item_emb, adj_indices, adj_values):
    rows = adj_indices[0]
    cols = adj_indices[1]
    nnz = cols.shape[0]
    step_edges = _NS * _SUPER
    nnz_pad = -(-nnz // step_edges) * step_edges
    pad = nnz_pad - nnz

    cols_p = jnp.pad(cols + (_PAD_HALF - _HALF) * (cols >= _HALF).astype(jnp.int32),
                     (0, pad))
    rows_p = jnp.pad(rows, (0, pad))
    vals_p = jnp.pad(adj_values, (0, pad))
    cols2 = cols_p.reshape(-1, _CHUNK)
    rows2 = rows_p.reshape(-1, _CHUNK)
    vals2 = vals_p.reshape(-1, _CHUNK)

    z = jnp.zeros((_PAD_HALF - _HALF, _EMB), jnp.float32)
    ego0 = jnp.concatenate([user_emb, z, item_emb, z], axis=0)

    layer = _make_layer()
    e1 = layer(ego0, cols2, rows2, vals2)
    e2 = layer(e1, cols2, rows2, vals2)
    e3 = layer(e2, cols2, rows2, vals2)
    fin = (e1 + e2 + e3) * jnp.float32(1.0 / 3.0)
    return fin[:_HALF], fin[_PAD_HALF:_PAD_HALF + _HALF]


# speedup vs baseline: 4.3034x; 1.7995x over previous
import functools

import jax
import jax.numpy as jnp
from jax import lax
from jax.experimental import pallas as pl
from jax.experimental.pallas import tpu as pltpu
from jax.experimental.pallas import tpu_sc as plsc

_EMB = 64
_HALF = 25000
_PAD_HALF = 25088
_EGO_PAD = 2 * _PAD_HALF
_NS = 16
_CH = 128
_SEX = 6
_BODY = 2 * _SEX
_N_BODY = 66
_TILE_CHUNKS = 134 * _SEX
_REAL_CHUNKS = 784
_ACC_SLICE = _PAD_HALF // _NS


def _spmm_body(ego, cols2, rows2, vals2, out,
               colv, rowv, rloc, valv, gath, acc,
               ga0, ga1, ga2, sc0, sc1, sc2, ed0, ed1):
    c = lax.axis_index("c")
    s = lax.axis_index("s")
    ga_sems = (ga0, ga1, ga2)
    sc_sems = (sc0, sc1, sc2)
    ed_sems = (ed0, ed1)
    zeros = jnp.zeros((16,), jnp.float32)
    c0 = c * _HALF
    tile_row0 = s * _TILE_CHUNKS

    def _fire_edges(q, p):
        r0 = tile_row0 + q * _SEX
        dsl = pl.ds(p * _SEX, _SEX)
        pltpu.async_copy(cols2.at[pl.ds(r0, _SEX)], colv.at[dsl], ed_sems[p])
        pltpu.async_copy(rows2.at[pl.ds(r0, _SEX)], rowv.at[dsl], ed_sems[p])
        pltpu.async_copy(vals2.at[pl.ds(r0, _SEX)], valv.at[dsl], ed_sems[p])

    def _wait_edges(p):
        dsl = pl.ds(p * _SEX, _SEX)
        pltpu.make_async_copy(cols2.at[pl.ds(0, _SEX)], colv.at[dsl], ed_sems[p]).wait()
        pltpu.make_async_copy(rows2.at[pl.ds(0, _SEX)], rowv.at[dsl], ed_sems[p]).wait()
        pltpu.make_async_copy(vals2.at[pl.ds(0, _SEX)], valv.at[dsl], ed_sems[p]).wait()

    def _fire_gather(erow, slot):
        pltpu.async_copy(ego.at[colv.at[erow]],
                         gath.at[pl.ds(slot * _CH, _CH)], ga_sems[slot])

    def _wait_gather(slot):
        pltpu.make_async_copy(ego.at[pl.ds(0, _CH)],
                              gath.at[pl.ds(slot * _CH, _CH)], ga_sems[slot]).wait()

    def _fire_scatter(slot):
        pltpu.async_copy(gath.at[pl.ds(slot * _CH, _CH)],
                         acc.at[rloc.at[slot]], sc_sems[slot], add=True)

    def _wait_scatter(slot):
        pltpu.make_async_copy(gath.at[pl.ds(slot * _CH, _CH)],
                              acc.at[pl.ds(0, _CH)], sc_sems[slot]).wait()

    def _chunk(ci):
        b = ci % 3
        _wait_gather(b)
        for jj in range(_CH // 16):
            sl = pl.ds(jj * 16, 16)
            r = rowv[ci, sl]
            rl = r - c0
            ok = (rl >= 0) & (rl < _HALF)
            trash = _HALF + (r & 63)
            rloc[b, sl] = jnp.where(ok, rl, trash)

        def _scale(i2, carry):
            vv = valv[ci, pl.ds(i2 * 16, 16)]
            for k in range(16):
                v = vv[k]
                row = b * _CH + i2 * 16 + k
                for q in range(_EMB // 16):
                    sl = pl.ds(q * 16, 16)
                    gath[row, sl] = gath[row, sl] * v
            return carry

        lax.fori_loop(0, _CH // 16, _scale, 0)
        _fire_scatter(b)
        pb = (ci + 2) % 3
        _wait_scatter(pb)
        _fire_gather(ci + 2 if ci + 2 < _BODY else ci + 2 - _BODY, pb)

    _fire_edges(0, 0)
    _fire_edges(1, 1)

    def _zero_gath(i, carry):
        for q in range(_EMB // 16):
            gath[i, pl.ds(q * 16, 16)] = zeros
        return carry

    lax.fori_loop(0, 3 * _CH, _zero_gath, 0, unroll=8)

    a0 = s * _ACC_SLICE
    for z in range(4):
        pltpu.sync_copy(gath.at[pl.ds(0, 3 * _CH)],
                        acc.at[pl.ds(a0 + z * 3 * _CH, 3 * _CH)])
    pltpu.sync_copy(gath.at[pl.ds(0, _ACC_SLICE - 12 * _CH)],
                    acc.at[pl.ds(a0 + 12 * _CH, _ACC_SLICE - 12 * _CH)])
    plsc.subcore_barrier()

    _wait_edges(0)
    _fire_gather(0, 0)
    _fire_gather(1, 1)
    for jj in range(_CH // 16):
        rloc[2, pl.ds(jj * 16, 16)] = jnp.zeros((16,), jnp.int32)
    _fire_scatter(2)

    def _step(m, carry):
        q0 = 2 * m
        for i in range(4):
            _chunk(i)
        _wait_edges(1)
        for i in range(4, 6):
            _chunk(i)
        _fire_edges(q0 + 2, 0)
        for i in range(6, 10):
            _chunk(i)
        _wait_edges(0)
        for i in range(10, 12):
            _chunk(i)
        _fire_edges(q0 + 3, 1)
        return carry

    lax.fori_loop(0, _N_BODY, _step, 0)

    _wait_scatter(2)
    _wait_gather(0)
    _wait_gather(1)
    _wait_edges(1)

    plsc.subcore_barrier()
    o0 = c * _PAD_HALF + a0
    h = _ACC_SLICE // 2
    pltpu.sync_copy(acc.at[pl.ds(a0, h)], out.at[pl.ds(o0, h)])
    pltpu.sync_copy(acc.at[pl.ds(a0 + h, h)], out.at[pl.ds(o0 + h, h)])


@functools.cache
def _make_layer():
    mesh = plsc.VectorSubcoreMesh(core_axis_name="c", subcore_axis_name="s")
    return pl.kernel(
        _spmm_body,
        out_type=jax.ShapeDtypeStruct((_EGO_PAD, _EMB), jnp.float32),
        mesh=mesh,
        compiler_params=pltpu.CompilerParams(use_tc_tiling_on_sc=False),
        scratch_types=[
            pltpu.VMEM((_BODY, _CH), jnp.int32),
            pltpu.VMEM((_BODY, _CH), jnp.int32),
            pltpu.VMEM((3, _CH), jnp.int32),
            pltpu.VMEM((_BODY, _CH), jnp.float32),
            pltpu.VMEM((3 * _CH, _EMB), jnp.float32),
            pltpu.VMEM_SHARED((_PAD_HALF, _EMB), jnp.float32),
            pltpu.SemaphoreType.DMA,
            pltpu.SemaphoreType.DMA,
            pltpu.SemaphoreType.DMA,
            pltpu.SemaphoreType.DMA,
            pltpu.SemaphoreType.DMA,
            pltpu.SemaphoreType.DMA,
            pltpu.SemaphoreType.DMA,
            pltpu.SemaphoreType.DMA,
        ],
    )


def kernel(user_emb, item_emb, adj_indices, adj_values):
    rows = adj_indices[0]
    cols = adj_indices[1]
    nnz = cols.shape[0]
    real_edges = _NS * _REAL_CHUNKS * _CH
    assert nnz <= real_edges

    def _layout(x):
        x = jnp.pad(x, (0, real_edges - nnz))
        x = x.reshape(_NS, _REAL_CHUNKS, _CH)
        x = jnp.pad(x, ((0, 0), (0, _TILE_CHUNKS - _REAL_CHUNKS), (0, 0)))
        return x.reshape(_NS * _TILE_CHUNKS, _CH)

    cols2 = _layout(cols + (_PAD_HALF - _HALF) * (cols >= _HALF).astype(jnp.int32))
    rows2 = _layout(rows)
    vals2 = _layout(adj_values)

    z = jnp.zeros((_PAD_HALF - _HALF, _EMB), jnp.float32)
    ego0 = jnp.concatenate([user_emb, z, item_emb, z], axis=0)

    layer = _make_layer()
    e1 = layer(ego0, cols2, rows2, vals2)
    e2 = layer(e1, cols2, rows2, vals2)
    e3 = layer(e2, cols2, rows2, vals2)
    fin = (e1 + e2 + e3) * jnp.float32(1.0 / 3.0)
    return fin[:_HALF], fin[_PAD_HALF:_PAD_HALF + _HALF]
